# baseline (speedup 1.0000x reference)
import functools

import jax
import jax.numpy as jnp
from jax import lax
from jax.experimental import pallas as pl
from jax.experimental.pallas import tpu as pltpu
from jax.experimental.pallas import tpu_sc as plsc

FEAT = 64
SCALE = 8.0
G = 128
KPB = 4
BLK = G * KPB
NC = 2
NS = 16
NW = NC * NS


def _emb_body(table_hbm, idx_hbm, out_hbm, idx_v, rows_v, gsem):
    wid = lax.axis_index("s") * NC + lax.axis_index("c")
    n_idx_rows = idx_hbm.shape[0]
    blocks_per_w = n_idx_rows // (KPB * NW)
    irow0 = wid * blocks_per_w * KPB

    def block(g, carry):
        irow = irow0 + g * KPB
        pltpu.sync_copy(idx_hbm.at[pl.ds(irow, KPB)], idx_v)
        copies = []
        for j in range(KPB):
            copies.append(
                pltpu.async_copy(
                    table_hbm.at[idx_v.at[j]],
                    rows_v.at[pl.ds(j * G, G)],
                    gsem,
                )
            )
        for c in copies:
            c.wait()
        def srow(r, c2):
            for col in range(FEAT // 16):
                sl = pl.ds(col * 16, 16)
                rows_v[r, sl] = rows_v[r, sl] * SCALE
            return c2
        lax.fori_loop(0, BLK, srow, 0)
        pltpu.sync_copy(rows_v, out_hbm.at[pl.ds(irow * G, BLK)])
        return carry

    lax.fori_loop(0, blocks_per_w, block, 0)


def kernel(token_idx, table):
    b, h = token_idx.shape
    n = b * h
    idx = token_idx.reshape(n // G, G).astype(jnp.int32)
    mesh = plsc.VectorSubcoreMesh(core_axis_name="c", subcore_axis_name="s")
    out = pl.kernel(
        _emb_body,
        out_type=jax.ShapeDtypeStruct((n, FEAT), jnp.float32),
        mesh=mesh,
        scratch_types=[
            pltpu.VMEM((KPB, G), jnp.int32),
            pltpu.VMEM((BLK, FEAT), jnp.float32),
            pltpu.SemaphoreType.DMA,
        ],
        compiler_params=pltpu.CompilerParams(use_tc_tiling_on_sc=False),
    )(table, idx)
    return out.reshape(b, h, FEAT)


# speedup vs baseline: 1.1225x; 1.1225x over previous
import jax
import jax.numpy as jnp
from jax import lax
from jax.experimental import pallas as pl
from jax.experimental.pallas import tpu as pltpu
from jax.experimental.pallas import tpu_sc as plsc

FEAT = 64
SCALE = 8.0
G = 128
KPB = 5
BLK = G * KPB
NC = 2
NS = 16
NW = NC * NS
UNROLL = 8


def _emb_body(table_hbm, idx_hbm, out_hbm,
              idx_v0, idx_v1, rows_v0, rows_v1,
              gsem0, gsem1, osem0, osem1):
    idx_v = (idx_v0, idx_v1)
    rows_v = (rows_v0, rows_v1)
    gsem = (gsem0, gsem1)
    osem = (osem0, osem1)

    wid = lax.axis_index("s") * NC + lax.axis_index("c")
    n_idx_rows = idx_hbm.shape[0]
    blocks = n_idx_rows // (KPB * NW)
    irow0 = wid * blocks * KPB

    def fire(b, irow):
        pltpu.sync_copy(idx_hbm.at[pl.ds(irow, KPB)], idx_v[b])
        for j in range(KPB):
            pltpu.async_copy(
                table_hbm.at[idx_v[b].at[j]],
                rows_v[b].at[pl.ds(j * G, G)],
                gsem[b],
            )

    def drain_gathers(b):
        for j in range(KPB):
            pltpu.make_async_copy(
                table_hbm.at[idx_v[b].at[j]],
                rows_v[b].at[pl.ds(j * G, G)],
                gsem[b],
            ).wait()

    def wait_store(b, irow):
        pltpu.make_async_copy(
            rows_v[b], out_hbm.at[pl.ds(irow * G, BLK)], osem[b]
        ).wait()

    def scale(b):
        def body(i, c):
            r0 = i * UNROLL
            for dr in range(UNROLL):
                for col in range(FEAT // 16):
                    sl = pl.ds(col * 16, 16)
                    rows_v[b][r0 + dr, sl] = rows_v[b][r0 + dr, sl] * SCALE
            return c
        lax.fori_loop(0, BLK // UNROLL, body, 0)

    fire(0, irow0)

    def pair(gp, carry):
        for b in range(2):
            g = gp * 2 + b
            irow = irow0 + g * KPB
            if b == 0:
                @pl.when(gp > 0)
                def _():
                    wait_store(1, irow - KPB)
            else:
                wait_store(0, irow - KPB)
            gn = lax.rem(g + 1, blocks)
            fire(1 - b, irow0 + gn * KPB)
            drain_gathers(b)
            scale(b)
            pltpu.async_copy(rows_v[b], out_hbm.at[pl.ds(irow * G, BLK)], osem[b])
        return carry

    lax.fori_loop(0, blocks // 2, pair, 0)

    drain_gathers(0)
    wait_store(1, irow0 + (blocks - 1) * KPB)


def kernel(token_idx, table):
    b, h = token_idx.shape
    n = b * h
    idx = token_idx.reshape(n // G, G).astype(jnp.int32)
    mesh = plsc.VectorSubcoreMesh(core_axis_name="c", subcore_axis_name="s")
    out = pl.kernel(
        _emb_body,
        out_type=jax.ShapeDtypeStruct((n, FEAT), jnp.float32),
        mesh=mesh,
        scratch_types=[
            pltpu.VMEM((KPB, G), jnp.int32),
            pltpu.VMEM((KPB, G), jnp.int32),
            pltpu.VMEM((BLK, FEAT), jnp.float32),
            pltpu.VMEM((BLK, FEAT), jnp.float32),
            pltpu.SemaphoreType.DMA,
            pltpu.SemaphoreType.DMA,
            pltpu.SemaphoreType.DMA,
            pltpu.SemaphoreType.DMA,
        ],
        compiler_params=pltpu.CompilerParams(use_tc_tiling_on_sc=False),
    )(table, idx)
    return out.reshape(b, h, FEAT)


# speedup vs baseline: 1.1253x; 1.0025x over previous
import jax
import jax.numpy as jnp
from jax import lax
from jax.experimental import pallas as pl
from jax.experimental.pallas import tpu as pltpu
from jax.experimental.pallas import tpu_sc as plsc

FEAT = 64
SCALE = 8.0
HIST = 200
G = 40
KB = 4
BLK = KB * HIST
NGB = BLK // G
NC = 2
NS = 16
NW = NC * NS
UNROLL = 8


def _emb_body(table_hbm, idx_hbm, out_hbm,
              idx_v0, idx_v1, rows_v0, rows_v1,
              gsem0, gsem1, osem0, osem1):
    idx_v = (idx_v0, idx_v1)
    rows_v = (rows_v0, rows_v1)
    gsem = (gsem0, gsem1)
    osem = (osem0, osem1)

    wid = lax.axis_index("s") * NC + lax.axis_index("c")
    batch = out_hbm.shape[0]
    blocks = batch // (KB * NW)
    be0 = wid * blocks * KB

    def fire(b, g_blk):
        pltpu.sync_copy(idx_hbm.at[pl.ds(g_blk * NGB, NGB)], idx_v[b])
        for j in range(NGB):
            pltpu.async_copy(
                table_hbm.at[idx_v[b].at[j]],
                rows_v[b].at[pl.ds(j * G, G)],
                gsem[b],
            )

    def drain_gathers(b):
        for j in range(NGB):
            pltpu.make_async_copy(
                table_hbm.at[idx_v[b].at[j]],
                rows_v[b].at[pl.ds(j * G, G)],
                gsem[b],
            ).wait()

    def store(b, be):
        for k in range(KB):
            pltpu.async_copy(
                rows_v[b].at[pl.ds(k * HIST, HIST)], out_hbm.at[be + k], osem[b]
            )

    def wait_store(b, be):
        for k in range(KB):
            pltpu.make_async_copy(
                rows_v[b].at[pl.ds(k * HIST, HIST)], out_hbm.at[be + k], osem[b]
            ).wait()

    def scale(b):
        def body(i, c):
            r0 = i * UNROLL
            for dr in range(UNROLL):
                for col in range(FEAT // 16):
                    sl = pl.ds(col * 16, 16)
                    rows_v[b][r0 + dr, sl] = rows_v[b][r0 + dr, sl] * SCALE
            return c
        lax.fori_loop(0, BLK // UNROLL, body, 0)

    blk0 = be0 // KB
    fire(0, blk0)

    def pair(gp, carry):
        for b in range(2):
            g = gp * 2 + b
            be = be0 + g * KB
            if b == 0:
                @pl.when(gp > 0)
                def _():
                    wait_store(1, be - KB)
            else:
                wait_store(0, be - KB)
            gn = lax.rem(g + 1, blocks)
            fire(1 - b, blk0 + gn)
            drain_gathers(b)
            scale(b)
            store(b, be)
        return carry

    lax.fori_loop(0, blocks // 2, pair, 0)

    drain_gathers(0)
    wait_store(1, be0 + (blocks - 1) * KB)


def kernel(token_idx, table):
    batch, hist = token_idx.shape
    idx = token_idx.reshape(batch * hist // G, G).astype(jnp.int32)
    mesh = plsc.VectorSubcoreMesh(core_axis_name="c", subcore_axis_name="s")
    out = pl.kernel(
        _emb_body,
        out_type=jax.ShapeDtypeStruct((batch, hist, FEAT), jnp.float32),
        mesh=mesh,
        scratch_types=[
            pltpu.VMEM((NGB, G), jnp.int32),
            pltpu.VMEM((NGB, G), jnp.int32),
            pltpu.VMEM((BLK, FEAT), jnp.float32),
            pltpu.VMEM((BLK, FEAT), jnp.float32),
            pltpu.SemaphoreType.DMA,
            pltpu.SemaphoreType.DMA,
            pltpu.SemaphoreType.DMA,
            pltpu.SemaphoreType.DMA,
        ],
        compiler_params=pltpu.CompilerParams(use_tc_tiling_on_sc=False),
    )(table, idx)
    return out


# speedup vs baseline: 1.3656x; 1.2135x over previous
import jax
import jax.numpy as jnp
from jax import lax
from jax.experimental import pallas as pl
from jax.experimental.pallas import tpu as pltpu
from jax.experimental.pallas import tpu_sc as plsc

FEAT = 64
WIDE = 128
SCALE = 8.0
HIST = 200
G = 40
NGB = 8
BLK = NGB * G
NC = 2
NS = 16
NW = NC * NS
UNROLL = 8


def _emb_body(table_hbm, idx_hbm, out_hbm,
              idx_v0, idx_v1, rows_v0, rows_v1,
              gsem0, gsem1, osem0, osem1):
    idx_v = (idx_v0, idx_v1)
    rows_v = (rows_v0, rows_v1)
    gsem = (gsem0, gsem1)
    osem = (osem0, osem1)

    wid = lax.axis_index("s") * NC + lax.axis_index("c")
    blocks = out_hbm.shape[0] // (BLK * NW)
    be0 = wid * blocks

    def fire(b, g_blk):
        pltpu.sync_copy(idx_hbm.at[pl.ds(g_blk * NGB, NGB)], idx_v[b])
        for j in range(NGB):
            pltpu.async_copy(
                table_hbm.at[idx_v[b].at[j]],
                rows_v[b].at[pl.ds(j * G, G)],
                gsem[b],
            )

    def drain_gathers(b):
        for j in range(NGB):
            pltpu.make_async_copy(
                table_hbm.at[idx_v[b].at[j]],
                rows_v[b].at[pl.ds(j * G, G)],
                gsem[b],
            ).wait()

    def store(b, g_blk):
        pltpu.async_copy(rows_v[b], out_hbm.at[pl.ds(g_blk * BLK, BLK)], osem[b])

    def wait_store(b, g_blk):
        pltpu.make_async_copy(
            rows_v[b], out_hbm.at[pl.ds(g_blk * BLK, BLK)], osem[b]
        ).wait()

    def scale(b):
        def body(i, c):
            r0 = i * UNROLL
            for dr in range(UNROLL):
                for col in range(FEAT // 16):
                    sl = pl.ds(col * 16, 16)
                    rows_v[b][r0 + dr, sl] = rows_v[b][r0 + dr, sl] * SCALE
            return c
        lax.fori_loop(0, BLK // UNROLL, body, 0)

    fire(0, be0)

    def pair(gp, carry):
        for b in range(2):
            g = gp * 2 + b
            blk = be0 + g
            if b == 0:
                @pl.when(gp > 0)
                def _():
                    wait_store(1, blk - 1)
            else:
                wait_store(0, blk - 1)
            gn = lax.rem(g + 1, blocks)
            fire(1 - b, be0 + gn)
            drain_gathers(b)
            scale(b)
            store(b, blk)
        return carry

    lax.fori_loop(0, blocks // 2, pair, 0)

    drain_gathers(0)
    wait_store(1, be0 + blocks - 1)


def kernel(token_idx, table):
    batch, hist = token_idx.shape
    n = batch * hist
    table_p = jnp.pad(table, ((0, 0), (0, WIDE - FEAT)))
    idx = token_idx.reshape(n // G, G).astype(jnp.int32)
    mesh = plsc.VectorSubcoreMesh(core_axis_name="c", subcore_axis_name="s")
    out = pl.kernel(
        _emb_body,
        out_type=jax.ShapeDtypeStruct((n, WIDE), jnp.float32),
        mesh=mesh,
        scratch_types=[
            pltpu.VMEM((NGB, G), jnp.int32),
            pltpu.VMEM((NGB, G), jnp.int32),
            pltpu.VMEM((BLK, WIDE), jnp.float32),
            pltpu.VMEM((BLK, WIDE), jnp.float32),
            pltpu.SemaphoreType.DMA,
            pltpu.SemaphoreType.DMA,
            pltpu.SemaphoreType.DMA,
            pltpu.SemaphoreType.DMA,
        ],
        compiler_params=pltpu.CompilerParams(use_tc_tiling_on_sc=True),
    )(table_p, idx)
    return out[:, :FEAT].reshape(batch, hist, FEAT)


# speedup vs baseline: 1.3852x; 1.0143x over previous
import jax
import jax.numpy as jnp
from jax import lax
from jax.experimental import pallas as pl
from jax.experimental.pallas import tpu as pltpu
from jax.experimental.pallas import tpu_sc as plsc

FEAT = 64
WIDE = 128
SCALE = 8.0
G = 32
NGB = 8
BLK = NGB * G
NC = 2
NS = 16
NW = NC * NS
UNROLL = 8


def _emb_body(table_hbm, idx_hbm, out_hbm,
              idx_v0, idx_v1, rows_v0, rows_v1, cmp_v,
              gsem0, gsem1, osem):
    idx_v = (idx_v0, idx_v1)
    rows_v = (rows_v0, rows_v1)
    gsem = (gsem0, gsem1)

    wid = lax.axis_index("s") * NC + lax.axis_index("c")
    blocks = out_hbm.shape[0] // (BLK * NW)
    be0 = wid * blocks

    def fire(b, g_blk):
        pltpu.sync_copy(idx_hbm.at[pl.ds(g_blk * NGB, NGB)], idx_v[b])
        for j in range(NGB):
            pltpu.async_copy(
                table_hbm.at[idx_v[b].at[j]],
                rows_v[b].at[pl.ds(j * G, G)],
                gsem[b],
            )

    def drain_gathers(b):
        for j in range(NGB):
            pltpu.make_async_copy(
                table_hbm.at[idx_v[b].at[j]],
                rows_v[b].at[pl.ds(j * G, G)],
                gsem[b],
            ).wait()

    def store(g_blk):
        pltpu.async_copy(cmp_v, out_hbm.at[pl.ds(g_blk * BLK, BLK)], osem)

    def wait_store(g_blk):
        pltpu.make_async_copy(
            cmp_v, out_hbm.at[pl.ds(g_blk * BLK, BLK)], osem
        ).wait()

    def scale(b):
        def body(i, c):
            r0 = i * UNROLL
            for dr in range(UNROLL):
                for col in range(FEAT // 16):
                    sl = pl.ds(col * 16, 16)
                    cmp_v[r0 + dr, sl] = rows_v[b][r0 + dr, sl] * SCALE
            return c
        lax.fori_loop(0, BLK // UNROLL, body, 0)

    fire(0, be0)

    def pair(gp, carry):
        for b in range(2):
            g = gp * 2 + b
            blk = be0 + g
            gn = lax.rem(g + 1, blocks)
            fire(1 - b, be0 + gn)
            drain_gathers(b)
            if b == 0:
                @pl.when(gp > 0)
                def _():
                    wait_store(blk - 1)
            else:
                wait_store(blk - 1)
            scale(b)
            store(blk)
        return carry

    lax.fori_loop(0, blocks // 2, pair, 0)

    drain_gathers(0)
    wait_store(be0 + blocks - 1)


def kernel(token_idx, table):
    batch, hist = token_idx.shape
    n = batch * hist
    table_p = jnp.pad(table, ((0, 0), (0, WIDE - FEAT)))
    idx = token_idx.reshape(n // G, G).astype(jnp.int32)
    mesh = plsc.VectorSubcoreMesh(core_axis_name="c", subcore_axis_name="s")
    out = pl.kernel(
        _emb_body,
        out_type=jax.ShapeDtypeStruct((n, FEAT), jnp.float32),
        mesh=mesh,
        scratch_types=[
            pltpu.VMEM((NGB, G), jnp.int32),
            pltpu.VMEM((NGB, G), jnp.int32),
            pltpu.VMEM((BLK, WIDE), jnp.float32),
            pltpu.VMEM((BLK, WIDE), jnp.float32),
            pltpu.VMEM((BLK, FEAT), jnp.float32),
            pltpu.SemaphoreType.DMA,
            pltpu.SemaphoreType.DMA,
            pltpu.SemaphoreType.DMA,
        ],
        compiler_params=pltpu.CompilerParams(use_tc_tiling_on_sc=True),
    )(table_p, idx)
    return out.reshape(batch, hist, FEAT)
